# baseline (speedup 1.0000x reference)
import functools

import jax
import jax.numpy as jnp
from jax import lax
from jax.experimental import pallas as pl
from jax.experimental.pallas import tpu as pltpu
from jax.experimental.pallas import tpu_sc as plsc

B = 4096
C = 1528
P = 4000

BB = 1024
NB = B // BB

NT = 12
GFLAT = NT * C * 128
HC = 768

NW = 16
CHUNK = 256
LANES = 16


def _tc_body(x_ref, t_ref, g_ref, base_ref):
    i = pl.program_id(0)
    x = x_ref[...]
    t = t_ref[...]
    th = jnp.tanh(0.5 * x)
    lg = -jnp.log(0.5 + 0.5 * jnp.abs(th))
    bce = jnp.maximum(x, 0.0) - x * t + lg
    part = jnp.sum(bce)
    s = 0.5 + 0.5 * th
    sb = s.astype(jnp.bfloat16)
    dn = (((1,), (1,)), ((), ()))
    top = lax.dot_general(sb[0:HC], sb, dn,
                          preferred_element_type=jnp.float32)
    bot = lax.dot_general(sb[HC:], sb[HC:], dn,
                          preferred_element_type=jnp.float32)
    cs = jnp.sum(s, axis=1, keepdims=True)

    nf = (C - HC) // 128
    stores = []
    for k in range(NT - 1):
        stores.append(((k, slice(0, HC), slice(None)),
                       top[:, k * 128:(k + 1) * 128]))
    stores.append(((NT - 1, slice(0, HC), slice(0, C - (NT - 1) * 128)),
                   top[:, (NT - 1) * 128:]))
    for k in range(nf):
        stores.append(((k, slice(HC, C), slice(None)),
                       bot[:, k * 128:(k + 1) * 128]))
    stores.append(((nf, slice(HC, C), slice(0, (C - HC) - nf * 128)),
                   bot[:, nf * 128:]))
    stores.append(((NT - 1, slice(None), slice(120, 121)), cs))

    @pl.when(i == 0)
    def _init():
        for sl, v in stores:
            g_ref[sl] = v
        base_ref[0, 0] = part

    @pl.when(i > 0)
    def _acc():
        for sl, v in stores:
            g_ref[sl] += v
        base_ref[0, 0] += part


def _tc_call(xt, tt):
    return pl.pallas_call(
        _tc_body,
        grid=(NB,),
        in_specs=[
            pl.BlockSpec((C, BB), lambda i: (0, i)),
            pl.BlockSpec((C, BB), lambda i: (0, i)),
        ],
        out_specs=[
            pl.BlockSpec((NT, C, 128), lambda i: (0, 0, 0)),
            pl.BlockSpec((1, 1), lambda i: (0, 0), memory_space=pltpu.SMEM),
        ],
        out_shape=[
            jax.ShapeDtypeStruct((NT, C, 128), jnp.float32),
            jax.ShapeDtypeStruct((1, 1), jnp.float32),
        ],
    )(xt, tt)


def _sc_body(g_hbm, l_hbm, r_hbm, out_hbm,
             l_v, r_v, idxh0_v, idxh1_v, idxc0_v, idxc1_v,
             valh0_v, valh1_v, valc0_v, valc1_v, acc_v, sem):
    wid = lax.axis_index("s")
    base = wid * CHUNK
    ngrp = CHUNK // LANES
    half_g = ngrp // 2
    nvalid = P - (NW - 1) * CHUNK

    @pl.when(wid < NW - 1)
    def _full():
        pltpu.sync_copy(l_hbm.at[pl.ds(base, CHUNK)], l_v)
        pltpu.sync_copy(r_hbm.at[pl.ds(base, CHUNK)], r_v)

    @pl.when(wid == NW - 1)
    def _tail():
        pltpu.sync_copy(l_hbm.at[pl.ds(base, nvalid)], l_v.at[pl.ds(0, nvalid)])
        pltpu.sync_copy(r_hbm.at[pl.ds(base, nvalid)], r_v.at[pl.ds(0, nvalid)])

    for j in range(1):
        sl = pl.ds(j * LANES, LANES)
        l = l_v[sl]
        r = r_v[sl]
        a = jnp.minimum(l, r)
        b2 = jnp.maximum(l, r)
        colw = b2 - jnp.where(a >= HC, HC, 0)
        idxh = ((colw >> 7) * C + a) * 128 + (colw & 127)
        idxc = ((NT - 1) * C + l) * 128 + 120
        hsl = pl.ds((j % half_g) * LANES, LANES)
        if j < half_g:
            idxh0_v[hsl] = idxh
            idxc0_v[hsl] = idxc
        else:
            idxh1_v[hsl] = idxh
            idxc1_v[hsl] = idxc

    @pl.when(wid == NW - 1)
    def _clamp():
        zero = jnp.zeros((LANES,), jnp.int32)
        for j in range(nvalid // LANES, ngrp):
            hsl = pl.ds((j % half_g) * LANES, LANES)
            if j < half_g:
                idxh0_v[hsl] = zero
                idxc0_v[hsl] = zero
            else:
                idxh1_v[hsl] = zero
                idxc1_v[hsl] = zero

    cps = [pltpu.async_copy(g_hbm.at[idxh0_v], valh0_v, sem),
           pltpu.async_copy(g_hbm.at[idxh1_v], valh1_v, sem),
           pltpu.async_copy(g_hbm.at[idxc0_v], valc0_v, sem),
           pltpu.async_copy(g_hbm.at[idxc1_v], valc1_v, sem)]
    for cp in cps:
        cp.wait()
    acc = jnp.zeros((LANES,), jnp.float32)
    lane = lax.iota(jnp.int32, LANES)
    for j in range(1):
        pos = base + j * LANES + lane
        hsl = pl.ds((j % half_g) * LANES, LANES)
        if j < half_g:
            v = valc0_v[hsl] - valh0_v[hsl]
        else:
            v = valc1_v[hsl] - valh1_v[hsl]
        acc = acc + jnp.where(pos < P, v, 0.0)
    acc_v[...] = acc
    pltpu.sync_copy(acc_v, out_hbm.at[wid])


def _sc_call(g_flat, l_idx, r_idx):
    mesh = plsc.VectorSubcoreMesh(core_axis_name="c", subcore_axis_name="s",
                                  num_cores=1)
    kern = functools.partial(
        pl.kernel,
        mesh=mesh,
        out_type=jax.ShapeDtypeStruct((NW, LANES), jnp.float32),
        scratch_types=[
            pltpu.VMEM((CHUNK,), jnp.int32),
            pltpu.VMEM((CHUNK,), jnp.int32),
            pltpu.VMEM((CHUNK // 2,), jnp.int32),
            pltpu.VMEM((CHUNK // 2,), jnp.int32),
            pltpu.VMEM((CHUNK // 2,), jnp.int32),
            pltpu.VMEM((CHUNK // 2,), jnp.int32),
            pltpu.VMEM((CHUNK // 2,), jnp.float32),
            pltpu.VMEM((CHUNK // 2,), jnp.float32),
            pltpu.VMEM((CHUNK // 2,), jnp.float32),
            pltpu.VMEM((CHUNK // 2,), jnp.float32),
            pltpu.VMEM((LANES,), jnp.float32),
            pltpu.SemaphoreType.DMA,
        ],
    )(_sc_body)
    return kern(g_flat, l_idx, r_idx)


def kernel(input, target, implication_filter_l, implication_filter_r):
    g3, base = _tc_call(input.T, target.T)
    partials = _sc_call(g3.reshape(-1),
                        implication_filter_l.astype(jnp.int32),
                        implication_filter_r.astype(jnp.int32))
    base_loss = base[0, 0] / (B * C)
    implication_loss = jnp.sum(partials) / B
    total = base_loss + 0.01 * implication_loss
    return (total, base_loss, implication_loss)


# speedup vs baseline: 1.0057x; 1.0057x over previous
import functools

import jax
import jax.numpy as jnp
from jax import lax
from jax.experimental import pallas as pl
from jax.experimental.pallas import tpu as pltpu
from jax.experimental.pallas import tpu_sc as plsc

B = 4096
C = 1528
P = 4000

BB = 1024
NB = B // BB

NT = 12
GFLAT = NT * C * 128
HC = 768

NW = 16
CHUNK = 256
LANES = 16


def _tc_body(x_ref, t_ref, g_ref, base_ref):
    i = pl.program_id(0)
    x = x_ref[...]
    t = t_ref[...]
    th = jnp.tanh(0.5 * x)
    lg = -jnp.log(0.5 + 0.5 * jnp.abs(th))
    bce = jnp.maximum(x, 0.0) - x * t + lg
    part = jnp.sum(bce)
    s = 0.5 + 0.5 * th
    sb = s.astype(jnp.bfloat16)
    dn = (((1,), (1,)), ((), ()))
    top = lax.dot_general(sb[0:HC], sb, dn,
                          preferred_element_type=jnp.float32)
    bot = lax.dot_general(sb[HC:], sb[HC:], dn,
                          preferred_element_type=jnp.float32)
    cs = jnp.sum(s, axis=1, keepdims=True)

    nf = (C - HC) // 128
    stores = []
    for k in range(NT - 1):
        stores.append(((k, slice(0, HC), slice(None)),
                       top[:, k * 128:(k + 1) * 128]))
    stores.append(((NT - 1, slice(0, HC), slice(0, C - (NT - 1) * 128)),
                   top[:, (NT - 1) * 128:]))
    for k in range(nf):
        stores.append(((k, slice(HC, C), slice(None)),
                       bot[:, k * 128:(k + 1) * 128]))
    stores.append(((nf, slice(HC, C), slice(0, (C - HC) - nf * 128)),
                   bot[:, nf * 128:]))
    stores.append(((NT - 1, slice(None), slice(120, 121)), cs))

    @pl.when(i == 0)
    def _init():
        for sl, v in stores:
            g_ref[sl] = v
        base_ref[0, 0] = part

    @pl.when(i > 0)
    def _acc():
        for sl, v in stores:
            g_ref[sl] += v
        base_ref[0, 0] += part


def _tc_call(xt, tt):
    return pl.pallas_call(
        _tc_body,
        grid=(NB,),
        in_specs=[
            pl.BlockSpec((C, BB), lambda i: (0, i)),
            pl.BlockSpec((C, BB), lambda i: (0, i)),
        ],
        out_specs=[
            pl.BlockSpec((NT, C, 128), lambda i: (0, 0, 0)),
            pl.BlockSpec((1, 1), lambda i: (0, 0), memory_space=pltpu.SMEM),
        ],
        out_shape=[
            jax.ShapeDtypeStruct((NT, C, 128), jnp.float32),
            jax.ShapeDtypeStruct((1, 1), jnp.float32),
        ],
    )(xt, tt)


def _sc_body(g_hbm, l_hbm, r_hbm, out_hbm,
             l_v, r_v, idxh0_v, idxh1_v, idxc0_v, idxc1_v,
             valh0_v, valh1_v, valc0_v, valc1_v, acc_v, sem):
    wid = lax.axis_index("s")
    base = wid * CHUNK
    ngrp = CHUNK // LANES
    half_g = ngrp // 2
    nvalid = P - (NW - 1) * CHUNK

    @pl.when(wid < NW - 1)
    def _full():
        pltpu.sync_copy(l_hbm.at[pl.ds(base, CHUNK)], l_v)
        pltpu.sync_copy(r_hbm.at[pl.ds(base, CHUNK)], r_v)

    @pl.when(wid == NW - 1)
    def _tail():
        pltpu.sync_copy(l_hbm.at[pl.ds(base, nvalid)], l_v.at[pl.ds(0, nvalid)])
        pltpu.sync_copy(r_hbm.at[pl.ds(base, nvalid)], r_v.at[pl.ds(0, nvalid)])

    for j in range(ngrp):
        sl = pl.ds(j * LANES, LANES)
        l = l_v[sl]
        r = r_v[sl]
        a = jnp.minimum(l, r)
        b2 = jnp.maximum(l, r)
        colw = b2 - jnp.where(a >= HC, HC, 0)
        idxh = ((colw >> 7) * C + a) * 128 + (colw & 127)
        idxc = ((NT - 1) * C + l) * 128 + 120
        hsl = pl.ds((j % half_g) * LANES, LANES)
        if j < half_g:
            idxh0_v[hsl] = idxh
            idxc0_v[hsl] = idxc
        else:
            idxh1_v[hsl] = idxh
            idxc1_v[hsl] = idxc

    @pl.when(wid == NW - 1)
    def _clamp():
        zero = jnp.zeros((LANES,), jnp.int32)
        for j in range(nvalid // LANES, ngrp):
            hsl = pl.ds((j % half_g) * LANES, LANES)
            if j < half_g:
                idxh0_v[hsl] = zero
                idxc0_v[hsl] = zero
            else:
                idxh1_v[hsl] = zero
                idxc1_v[hsl] = zero

    cps = [pltpu.async_copy(g_hbm.at[idxh0_v], valh0_v, sem),
           pltpu.async_copy(g_hbm.at[idxh1_v], valh1_v, sem),
           pltpu.async_copy(g_hbm.at[idxc0_v], valc0_v, sem),
           pltpu.async_copy(g_hbm.at[idxc1_v], valc1_v, sem)]
    for cp in cps:
        cp.wait()
    acc = jnp.zeros((LANES,), jnp.float32)
    lane = lax.iota(jnp.int32, LANES)
    for j in range(ngrp):
        pos = base + j * LANES + lane
        hsl = pl.ds((j % half_g) * LANES, LANES)
        if j < half_g:
            v = valc0_v[hsl] - valh0_v[hsl]
        else:
            v = valc1_v[hsl] - valh1_v[hsl]
        acc = acc + jnp.where(pos < P, v, 0.0)
    acc_v[...] = acc
    pltpu.sync_copy(acc_v, out_hbm.at[wid])


def _sc_call(g_flat, l_idx, r_idx):
    mesh = plsc.VectorSubcoreMesh(core_axis_name="c", subcore_axis_name="s",
                                  num_cores=1)
    kern = functools.partial(
        pl.kernel,
        mesh=mesh,
        out_type=jax.ShapeDtypeStruct((NW, LANES), jnp.float32),
        scratch_types=[
            pltpu.VMEM((CHUNK,), jnp.int32),
            pltpu.VMEM((CHUNK,), jnp.int32),
            pltpu.VMEM((CHUNK // 2,), jnp.int32),
            pltpu.VMEM((CHUNK // 2,), jnp.int32),
            pltpu.VMEM((CHUNK // 2,), jnp.int32),
            pltpu.VMEM((CHUNK // 2,), jnp.int32),
            pltpu.VMEM((CHUNK // 2,), jnp.float32),
            pltpu.VMEM((CHUNK // 2,), jnp.float32),
            pltpu.VMEM((CHUNK // 2,), jnp.float32),
            pltpu.VMEM((CHUNK // 2,), jnp.float32),
            pltpu.VMEM((LANES,), jnp.float32),
            pltpu.SemaphoreType.DMA,
        ],
    )(_sc_body)
    return kern(g_flat, l_idx, r_idx)


def kernel(input, target, implication_filter_l, implication_filter_r):
    g3, base = _tc_call(input.T, target.T)
    partials = _sc_call(g3.reshape(-1),
                        implication_filter_l.astype(jnp.int32),
                        implication_filter_r.astype(jnp.int32))
    base_loss = base[0, 0] / (B * C)
    implication_loss = jnp.sum(partials) / B
    total = base_loss + 0.01 * implication_loss
    return (total, base_loss, implication_loss)


# speedup vs baseline: 1.0139x; 1.0081x over previous
import functools

import jax
import jax.numpy as jnp
from jax import lax
from jax.experimental import pallas as pl
from jax.experimental.pallas import tpu as pltpu
from jax.experimental.pallas import tpu_sc as plsc

B = 4096
C = 1528
P = 4000

BB = 1024
NB = B // BB

NT = 12
HC = 768

NW = 16
CHUNK = 256
LANES = 16


def _tc_body(x_ref, t_ref, g_ref, base_ref):
    i = pl.program_id(0)
    x = x_ref[...]
    t = t_ref[...]
    th = jnp.tanh(0.5 * x)
    lg = -jnp.log(0.5 + 0.5 * jnp.abs(th))
    bce = jnp.maximum(x, 0.0) - x * t + lg
    part = jnp.sum(bce)
    s = 0.5 + 0.5 * th
    sb = s.astype(jnp.bfloat16)
    dn = (((1,), (1,)), ((), ()))
    top = lax.dot_general(sb[0:HC], sb, dn,
                          preferred_element_type=jnp.float32)
    bot = lax.dot_general(sb[HC:], sb[HC:], dn,
                          preferred_element_type=jnp.float32)
    cs = jnp.sum(s, axis=1, keepdims=True)

    nf = (C - HC) // 128
    stores = []
    for k in range(NT - 1):
        stores.append(((k, slice(0, HC), slice(None)),
                       top[:, k * 128:(k + 1) * 128]))
    stores.append(((NT - 1, slice(0, HC), slice(0, C - (NT - 1) * 128)),
                   top[:, (NT - 1) * 128:]))
    for k in range(nf):
        stores.append(((k, slice(HC, C), slice(None)),
                       bot[:, k * 128:(k + 1) * 128]))
    stores.append(((nf, slice(HC, C), slice(0, (C - HC) - nf * 128)),
                   bot[:, nf * 128:]))
    stores.append(((NT - 1, slice(None), slice(120, 121)), cs))

    @pl.when(i == 0)
    def _init():
        for sl, v in stores:
            g_ref[sl] = v
        base_ref[0, 0] = part

    @pl.when(i > 0)
    def _acc():
        for sl, v in stores:
            g_ref[sl] += v
        base_ref[0, 0] += part


def _tc_call(xt, tt):
    return pl.pallas_call(
        _tc_body,
        grid=(NB,),
        in_specs=[
            pl.BlockSpec((C, BB), lambda i: (0, i)),
            pl.BlockSpec((C, BB), lambda i: (0, i)),
        ],
        out_specs=[
            pl.BlockSpec((NT, C, 128), lambda i: (0, 0, 0)),
            pl.BlockSpec((1, 1), lambda i: (0, 0), memory_space=pltpu.SMEM),
        ],
        out_shape=[
            jax.ShapeDtypeStruct((NT, C, 128), jnp.float32),
            jax.ShapeDtypeStruct((1, 1), jnp.float32),
        ],
    )(xt, tt)


def _sc_body(g_hbm, l_hbm, r_hbm, out_hbm,
             l_v, r_v, idxh0_v, idxh1_v, idxc0_v, idxc1_v,
             valh0_v, valh1_v, valc0_v, valc1_v, acc_v, sem):
    wid = lax.axis_index("s")
    base = wid * CHUNK
    ngrp = CHUNK // LANES
    half_g = ngrp // 2
    nvalid = P - (NW - 1) * CHUNK

    @pl.when(wid < NW - 1)
    def _full():
        pltpu.sync_copy(l_hbm.at[pl.ds(base, CHUNK)], l_v)
        pltpu.sync_copy(r_hbm.at[pl.ds(base, CHUNK)], r_v)

    @pl.when(wid == NW - 1)
    def _tail():
        pltpu.sync_copy(l_hbm.at[pl.ds(base, nvalid)], l_v.at[pl.ds(0, nvalid)])
        pltpu.sync_copy(r_hbm.at[pl.ds(base, nvalid)], r_v.at[pl.ds(0, nvalid)])

    for j in range(ngrp):
        sl = pl.ds(j * LANES, LANES)
        l = l_v[sl]
        r = r_v[sl]
        a = jnp.minimum(l, r)
        b2 = jnp.maximum(l, r)
        colw = b2 - jnp.where(a >= HC, HC, 0)
        idxh = ((colw >> 7) * C + a) * 128 + (colw & 127)
        idxc = ((NT - 1) * C + l) * 128 + 120
        hsl = pl.ds((j % half_g) * LANES, LANES)
        if j < half_g:
            idxh0_v[hsl] = idxh
            idxc0_v[hsl] = idxc
        else:
            idxh1_v[hsl] = idxh
            idxc1_v[hsl] = idxc

    @pl.when(wid == NW - 1)
    def _clamp():
        zero = jnp.zeros((LANES,), jnp.int32)
        for j in range(nvalid // LANES, ngrp):
            hsl = pl.ds((j % half_g) * LANES, LANES)
            if j < half_g:
                idxh0_v[hsl] = zero
                idxc0_v[hsl] = zero
            else:
                idxh1_v[hsl] = zero
                idxc1_v[hsl] = zero

    cps = [pltpu.async_copy(g_hbm.at[idxh0_v], valh0_v, sem),
           pltpu.async_copy(g_hbm.at[idxh1_v], valh1_v, sem),
           pltpu.async_copy(g_hbm.at[idxc0_v], valc0_v, sem),
           pltpu.async_copy(g_hbm.at[idxc1_v], valc1_v, sem)]
    for cp in cps:
        cp.wait()
    acc = jnp.zeros((LANES,), jnp.float32)
    lane = lax.iota(jnp.int32, LANES)
    for j in range(ngrp):
        pos = base + j * LANES + lane
        hsl = pl.ds((j % half_g) * LANES, LANES)
        if j < half_g:
            v = valc0_v[hsl] - valh0_v[hsl]
        else:
            v = valc1_v[hsl] - valh1_v[hsl]
        acc = acc + jnp.where(pos < P, v, 0.0)
    acc_v[...] = acc
    pltpu.sync_copy(acc_v, out_hbm.at[wid])


def _sc_call(g_flat, l_idx, r_idx):
    mesh = plsc.VectorSubcoreMesh(core_axis_name="c", subcore_axis_name="s",
                                  num_cores=1)
    kern = functools.partial(
        pl.kernel,
        mesh=mesh,
        out_type=jax.ShapeDtypeStruct((NW, LANES), jnp.float32),
        scratch_types=[
            pltpu.VMEM((CHUNK,), jnp.int32),
            pltpu.VMEM((CHUNK,), jnp.int32),
            pltpu.VMEM((CHUNK // 2,), jnp.int32),
            pltpu.VMEM((CHUNK // 2,), jnp.int32),
            pltpu.VMEM((CHUNK // 2,), jnp.int32),
            pltpu.VMEM((CHUNK // 2,), jnp.int32),
            pltpu.VMEM((CHUNK // 2,), jnp.float32),
            pltpu.VMEM((CHUNK // 2,), jnp.float32),
            pltpu.VMEM((CHUNK // 2,), jnp.float32),
            pltpu.VMEM((CHUNK // 2,), jnp.float32),
            pltpu.VMEM((LANES,), jnp.float32),
            pltpu.SemaphoreType.DMA,
        ],
    )(_sc_body)
    return kern(g_flat, l_idx, r_idx)


def kernel(input, target, implication_filter_l, implication_filter_r):
    g3, base = _tc_call(input.T, target.T)
    partials = _sc_call(g3.reshape(-1),
                        implication_filter_l.astype(jnp.int32),
                        implication_filter_r.astype(jnp.int32))
    base_loss = base[0, 0] / (B * C)
    implication_loss = jnp.sum(partials) / B
    total = base_loss + 0.01 * implication_loss
    return (total, base_loss, implication_loss)
